# baseline (speedup 1.0000x reference)
import functools

import jax
import jax.numpy as jnp
from jax import lax
from jax.experimental import pallas as pl
from jax.experimental.pallas import tpu as pltpu
from jax.experimental.pallas import tpu_sc as plsc


def _make_gather(n_workers, n_chunks, chunk, d_pad):
    mesh = plsc.VectorSubcoreMesh(core_axis_name="c", subcore_axis_name="s")
    per_w = n_chunks * chunk

    @functools.partial(
        pl.kernel,
        out_type=jax.ShapeDtypeStruct((n_workers * per_w, d_pad), jnp.float32),
        mesh=mesh,
        scratch_types=[
            pltpu.VMEM((n_chunks, chunk), jnp.int32),
            pltpu.VMEM((chunk, d_pad), jnp.float32),
            pltpu.VMEM((chunk, d_pad), jnp.float32),
            pltpu.SemaphoreType.DMA,
            pltpu.SemaphoreType.DMA,
            pltpu.SemaphoreType.DMA,
            pltpu.SemaphoreType.DMA,
        ],
    )
    def glove_gather(idx_hbm, table_hbm, out_hbm,
                     idx_v, buf0, buf1, gs0, gs1, ws0, ws1):
        n_cores = mesh.num_cores
        wid = lax.axis_index("s") * n_cores + lax.axis_index("c")
        row_base = wid * per_w
        bufs = (buf0, buf1)
        gsem = (gs0, gs1)
        wsem = (ws0, ws1)

        pltpu.sync_copy(idx_hbm.at[wid], idx_v)

        def fire_gather(g, bi):
            pltpu.async_copy(table_hbm.at[idx_v.at[g]], bufs[bi], gsem[bi])

        def wait_gather(bi):
            pltpu.make_async_copy(
                table_hbm.at[idx_v.at[0]], bufs[bi], gsem[bi]).wait()

        def fire_write(g, bi):
            pltpu.async_copy(
                bufs[bi], out_hbm.at[pl.ds(row_base + g * chunk, chunk)],
                wsem[bi])

        def wait_write(bi):
            pltpu.make_async_copy(
                bufs[bi], out_hbm.at[pl.ds(row_base, chunk)], wsem[bi]).wait()

        def body(i, carry):
            g0 = 2 * i
            g1 = g0 + 1

            @pl.when(i > 0)
            def _():
                wait_write(0)

            fire_gather(g0, 0)

            @pl.when(i > 0)
            def _():
                wait_write(1)

            fire_gather(g1, 1)
            wait_gather(0)
            fire_write(g0, 0)
            wait_gather(1)
            fire_write(g1, 1)
            return carry

        lax.fori_loop(0, n_chunks // 2, body, 0)
        wait_write(0)
        wait_write(1)

    return glove_gather


def kernel(x, table):
    b, s = x.shape
    v, d = table.shape
    d_pad = 384
    n = b * s
    n_workers = 32
    chunk = 128
    per_w = n // n_workers
    n_chunks = per_w // chunk
    idx = x.reshape(n_workers, n_chunks, chunk).astype(jnp.int32)
    table_pad = jnp.zeros((v, d_pad), jnp.float32).at[:, :d].set(table)
    out = _make_gather(n_workers, n_chunks, chunk, d_pad)(idx, table_pad)
    one = lax.optimization_barrier(jnp.float32(1.0))
    return (out[:, :d] * one).reshape(b, s, d)


# speedup vs baseline: 1.3771x; 1.3771x over previous
import functools

import jax
import jax.numpy as jnp
from jax import lax
from jax.experimental import pallas as pl
from jax.experimental.pallas import tpu as pltpu
from jax.experimental.pallas import tpu_sc as plsc


def _make_gather(n_workers, n_chunks, chunk, d_pad):
    mesh = plsc.VectorSubcoreMesh(core_axis_name="c", subcore_axis_name="s")
    per_w = n_chunks * chunk

    @functools.partial(
        pl.kernel,
        out_type=jax.ShapeDtypeStruct((n_workers * per_w, d_pad), jnp.float32),
        mesh=mesh,
        scratch_types=[
            pltpu.VMEM((n_chunks, chunk), jnp.int32),
            pltpu.VMEM((chunk, d_pad), jnp.float32),
            pltpu.VMEM((chunk, d_pad), jnp.float32),
            pltpu.SemaphoreType.DMA,
            pltpu.SemaphoreType.DMA,
            pltpu.SemaphoreType.DMA,
            pltpu.SemaphoreType.DMA,
        ],
    )
    def glove_gather(idx_hbm, table_hbm, out_hbm,
                     idx_v, buf0, buf1, gs0, gs1, ws0, ws1):
        n_cores = mesh.num_cores
        wid = lax.axis_index("s") * n_cores + lax.axis_index("c")
        row_base = wid * per_w
        bufs = (buf0, buf1)
        gsem = (gs0, gs1)
        wsem = (ws0, ws1)

        pltpu.sync_copy(idx_hbm.at[wid], idx_v)

        def fire_gather(g, bi):
            pltpu.async_copy(table_hbm.at[idx_v.at[g]], bufs[bi], gsem[bi])

        def wait_gather(bi):
            pltpu.make_async_copy(
                table_hbm.at[idx_v.at[0]], bufs[bi], gsem[bi]).wait()

        def fire_write(g, bi):
            pltpu.async_copy(
                bufs[bi], out_hbm.at[pl.ds(row_base + g * chunk, chunk)],
                wsem[bi])

        def wait_write(bi):
            pltpu.make_async_copy(
                bufs[bi], out_hbm.at[pl.ds(row_base, chunk)], wsem[bi]).wait()

        def body(i, carry):
            g0 = 2 * i
            g1 = g0 + 1

            @pl.when(i > 0)
            def _():
                wait_write(0)

            fire_gather(g0, 0)

            @pl.when(i > 0)
            def _():
                wait_write(1)

            fire_gather(g1, 1)
            wait_gather(0)
            fire_write(g0, 0)
            wait_gather(1)
            fire_write(g1, 1)
            return carry

        lax.fori_loop(0, n_chunks // 2, body, 0)
        wait_write(0)
        wait_write(1)

    return glove_gather


def kernel(x, table):
    b, s = x.shape
    v, d = table.shape
    d_pad = 384
    n = b * s
    n_workers = 32
    chunk = 128
    per_w = n // n_workers
    n_chunks = per_w // chunk
    idx = x.reshape(n_workers, n_chunks, chunk).astype(jnp.int32)
    table_pad = jnp.pad(table, ((0, 0), (0, d_pad - d)))
    out = _make_gather(n_workers, n_chunks, chunk, d_pad)(idx, table_pad)
    return out[:, :d].reshape(b, s, d)


# speedup vs baseline: 1.3784x; 1.0009x over previous
import functools

import jax
import jax.numpy as jnp
from jax import lax
from jax.experimental import pallas as pl
from jax.experimental.pallas import tpu as pltpu
from jax.experimental.pallas import tpu_sc as plsc

_NBUF = 3


def _make_gather(n_workers, n_chunks, chunk, d_pad):
    mesh = plsc.VectorSubcoreMesh(core_axis_name="c", subcore_axis_name="s")
    per_w = n_chunks * chunk

    @functools.partial(
        pl.kernel,
        out_type=jax.ShapeDtypeStruct((n_workers * per_w, d_pad), jnp.float32),
        mesh=mesh,
        scratch_types=(
            [pltpu.VMEM((n_chunks, chunk), jnp.int32)]
            + [pltpu.VMEM((chunk, d_pad), jnp.float32)] * _NBUF
            + [pltpu.SemaphoreType.DMA] * (2 * _NBUF)
        ),
    )
    def glove_gather(idx_hbm, table_hbm, out_hbm, idx_v, *bufs_sems):
        bufs = bufs_sems[:_NBUF]
        gsem = bufs_sems[_NBUF:2 * _NBUF]
        wsem = bufs_sems[2 * _NBUF:]
        n_cores = mesh.num_cores
        wid = lax.axis_index("s") * n_cores + lax.axis_index("c")
        row_base = wid * per_w

        pltpu.sync_copy(idx_hbm.at[wid], idx_v)

        def fire_gather(g, bi):
            pltpu.async_copy(table_hbm.at[idx_v.at[g]], bufs[bi], gsem[bi])

        def wait_gather(bi):
            pltpu.make_async_copy(
                table_hbm.at[idx_v.at[0]], bufs[bi], gsem[bi]).wait()

        def fire_write(g, bi):
            pltpu.async_copy(
                bufs[bi], out_hbm.at[pl.ds(row_base + g * chunk, chunk)],
                wsem[bi])

        def wait_write(bi):
            pltpu.make_async_copy(
                bufs[bi], out_hbm.at[pl.ds(row_base, chunk)], wsem[bi]).wait()

        def body(i, carry):
            for j in range(_NBUF):
                g = _NBUF * i + j

                @pl.when(i > 0)
                def _():
                    wait_write(j)

                fire_gather(g, j)
                pj = (j - 1) % _NBUF
                if j > 0:
                    wait_gather(pj)
                    fire_write(g - 1, pj)
                else:
                    @pl.when(i > 0)
                    def _():
                        wait_gather(pj)
                        fire_write(g - 1, pj)
            return carry

        lax.fori_loop(0, n_chunks // _NBUF, body, 0)
        wait_gather(_NBUF - 1)
        fire_write(n_chunks - 1, _NBUF - 1)
        for j in range(_NBUF):
            wait_write(j)

    return glove_gather


def kernel(x, table):
    b, s = x.shape
    v, d = table.shape
    d_pad = 384
    n = b * s
    n_workers = 32
    chunk = 64
    per_w = n // n_workers
    n_chunks = per_w // chunk
    idx = x.reshape(n_workers, n_chunks, chunk).astype(jnp.int32)
    table_pad = jnp.pad(table, ((0, 0), (0, d_pad - d)))
    out = _make_gather(n_workers, n_chunks, chunk, d_pad)(idx, table_pad)
    return out[:, :d].reshape(b, s, d)
